# Initial kernel scaffold; baseline (speedup 1.0000x reference)
import jax
import jax.numpy as jnp
from jax.experimental import pallas as pl


def kernel(processor_features, batch_size, edge_index, edge_attr, eeW0, eeb0, eeW1, eeb1, eeW2, eeb2, eeg, eebe, epW0, epb0, epW1, epb1, epW2, epb2, epg, epbe, np_W0, np_b0, np_W1, np_b1, np_W2, np_b2, np_g, np_be, decW0, decb0, decW1, decb1, decW2, decb2):
    raise NotImplementedError("write your pallas kernel here")


# speedup vs baseline: 7.0945x; 7.0945x over previous
import functools

import jax
import jax.numpy as jnp
from jax import lax
from jax.experimental import pallas as pl
from jax.experimental.pallas import tpu as pltpu
from jax.experimental.pallas import tpu_sc as plsc

NUM_H3 = 5882
HID = 128
OUT_DIM = 78
DEG = 7

NW = 32
CHUNK = 128
NBUF = 4


def _ln(x, g, b, eps=1e-5):
    m = jnp.mean(x, axis=-1, keepdims=True)
    xc = x - m
    v = jnp.mean(xc * xc, axis=-1, keepdims=True)
    return xc * lax.rsqrt(v + eps) * g + b


def _prep_body(pf_ref, w_ref, out_ref):
    out_ref[...] = jnp.dot(pf_ref[...], w_ref[...],
                           preferred_element_type=jnp.float32)


def _sc_gather(table, idx3):
    n_chunks = idx3.shape[1]
    b_per_w = n_chunks * CHUNK
    B = NW * b_per_w
    D = table.shape[1]
    mesh = plsc.VectorSubcoreMesh(core_axis_name="c", subcore_axis_name="s")

    def body(table_hbm, idx_hbm, out_hbm, idx_v, bufs, sems):
        wid = lax.axis_index("s") * 2 + lax.axis_index("c")
        base = wid * b_per_w
        pltpu.sync_copy(idx_hbm.at[wid], idx_v)
        for b in range(NBUF):
            pltpu.async_copy(table_hbm.at[idx_v.at[b]], bufs[b], sems[b])

        def outer(o, carry):
            for b in range(NBUF):
                j = o * NBUF + b
                pltpu.make_async_copy(table_hbm.at[idx_v.at[b]], bufs[b],
                                      sems[b]).wait()
                pltpu.sync_copy(bufs[b],
                                out_hbm.at[pl.ds(base + j * CHUNK, CHUNK)])
                jn = j + NBUF

                @pl.when(jn < n_chunks)
                def _():
                    pltpu.async_copy(table_hbm.at[idx_v.at[jn]], bufs[b],
                                     sems[b])
            return carry

        lax.fori_loop(0, n_chunks // NBUF, outer, 0)

    run = pl.kernel(
        body,
        out_type=jax.ShapeDtypeStruct((B, D), table.dtype),
        mesh=mesh,
        scratch_types=dict(
            idx_v=pltpu.VMEM((n_chunks, CHUNK), jnp.int32),
            bufs=[pltpu.VMEM((CHUNK, D), table.dtype) for _ in range(NBUF)],
            sems=[pltpu.SemaphoreType.DMA for _ in range(NBUF)],
        ),
    )
    return run(table, idx3)


def _main_body(ea_ref, g_ref, eeW0, eeb0, eeW1, eeb1, eeW2, eeb2, eeg, eebe,
               epW0c, epb0, epW1, epb1, epW2, epb2, epg, epbe,
               npW0b, np_b0, np_W1, np_b1, np_W2, np_b2, np_g, np_be,
               decW0, decb0, decW1, decb1, decW2, decb2, out_ref):
    f32 = jnp.float32
    be = ea_ref.shape[0]
    h = jnp.maximum(jnp.dot(ea_ref[...], eeW0[...],
                            preferred_element_type=f32) + eeb0[...], 0.0)
    h = jnp.maximum(jnp.dot(h, eeW1[...],
                            preferred_element_type=f32) + eeb1[...], 0.0)
    e0 = jnp.dot(h, eeW2[...], preferred_element_type=f32) + eeb2[...]
    e0 = _ln(e0, eeg[...], eebe[...])
    ec = jnp.dot(e0, epW0c[...], preferred_element_type=f32) + epb0[...]
    bs = g_ref.shape[0]
    for b in range(bs):
        hb = jnp.maximum(g_ref[b] + ec, 0.0)
        hb = jnp.maximum(jnp.dot(hb, epW1[...],
                                 preferred_element_type=f32) + epb1[...], 0.0)
        eb = jnp.dot(hb, epW2[...], preferred_element_type=f32) + epb2[...]
        eb = _ln(eb, epg[...], epbe[...]) + e0
        agg = eb.reshape(be // DEG, DEG, HID).sum(axis=1)
        nh = jnp.maximum(jnp.dot(agg, npW0b[...],
                                 preferred_element_type=f32) + np_b0[...], 0.0)
        nh = jnp.maximum(jnp.dot(nh, np_W1[...],
                                 preferred_element_type=f32) + np_b1[...], 0.0)
        xb = jnp.dot(nh, np_W2[...], preferred_element_type=f32) + np_b2[...]
        xb = _ln(xb, np_g[...], np_be[...])
        o = jnp.maximum(jnp.dot(xb, decW0[...],
                                preferred_element_type=f32) + decb0[...], 0.0)
        o = jnp.maximum(jnp.dot(o, decW1[...],
                                preferred_element_type=f32) + decb1[...], 0.0)
        o = jnp.dot(o, decW2[...], preferred_element_type=f32) + decb2[...]
        out_ref[b] = o


def kernel(processor_features, batch_size, edge_index, edge_attr,
           eeW0, eeb0, eeW1, eeb1, eeW2, eeb2, eeg, eebe,
           epW0, epb0, epW1, epb1, epW2, epb2, epg, epbe,
           np_W0, np_b0, np_W1, np_b1, np_W2, np_b2, np_g, np_be,
           decW0, decb0, decW1, decb1, decW2, decb2):
    del batch_size
    pf = processor_features
    bs = pf.shape[0] // NUM_H3
    E0 = edge_attr.shape[0]
    num_latlons = E0 // DEG

    epW0a = epW0[:HID]
    prep = pl.pallas_call(
        _prep_body,
        out_shape=jax.ShapeDtypeStruct((bs * NUM_H3, HID), jnp.float32),
    )
    table = prep(pf, epW0a)

    src = edge_index[0]
    idx = jnp.concatenate([src + b * NUM_H3 for b in range(bs)])
    n_chunks = idx.shape[0] // (NW * CHUNK)
    idx3 = idx.reshape(NW, n_chunks, CHUNK)
    g = _sc_gather(table, idx3).reshape(bs, E0, HID)

    BE = 7168
    grid = (E0 // BE,)
    full = lambda a: pl.BlockSpec(a.shape, lambda i: (0,) * a.ndim)
    weights = (eeW0, eeb0, eeW1, eeb1, eeW2, eeb2, eeg, eebe,
               epW0[2 * HID:], epb0, epW1, epb1, epW2, epb2, epg, epbe,
               np_W0[HID:], np_b0, np_W1, np_b1, np_W2, np_b2, np_g, np_be,
               decW0, decb0, decW1, decb1, decW2, decb2)
    out = pl.pallas_call(
        _main_body,
        grid=grid,
        in_specs=[
            pl.BlockSpec((BE, 2), lambda i: (i, 0)),
            pl.BlockSpec((bs, BE, HID), lambda i: (0, i, 0)),
        ] + [full(w) for w in weights],
        out_specs=pl.BlockSpec((bs, BE // DEG, OUT_DIM), lambda i: (0, i, 0)),
        out_shape=jax.ShapeDtypeStruct((bs, num_latlons, OUT_DIM),
                                       jnp.float32),
        compiler_params=pltpu.CompilerParams(
            dimension_semantics=("arbitrary",)),
    )(edge_attr, g, *weights)
    return out


# speedup vs baseline: 7.9219x; 1.1166x over previous
import functools

import jax
import jax.numpy as jnp
from jax import lax
from jax.experimental import pallas as pl
from jax.experimental.pallas import tpu as pltpu
from jax.experimental.pallas import tpu_sc as plsc

NUM_H3 = 5882
HID = 128
OUT_DIM = 78
DEG = 7

NW = 32
CHUNK = 128
NBUF = 4


def _ln(x, g, b, eps=1e-5):
    m = jnp.mean(x, axis=-1, keepdims=True)
    xc = x - m
    v = jnp.mean(xc * xc, axis=-1, keepdims=True)
    return xc * lax.rsqrt(v + eps) * g + b


def _prep_body(pf_ref, w_ref, out_ref):
    out_ref[...] = jnp.dot(pf_ref[...], w_ref[...],
                           preferred_element_type=jnp.float32)


def _sc_gather(table, idx3):
    n_chunks = idx3.shape[1]
    b_per_w = n_chunks * CHUNK
    B = NW * b_per_w
    D = table.shape[1]
    mesh = plsc.VectorSubcoreMesh(core_axis_name="c", subcore_axis_name="s")

    def body(table_hbm, idx_hbm, out_hbm, idx_v, bufs, sems):
        wid = lax.axis_index("s") * 2 + lax.axis_index("c")
        base = wid * b_per_w
        pltpu.sync_copy(idx_hbm.at[wid], idx_v)
        for b in range(NBUF):
            pltpu.async_copy(table_hbm.at[idx_v.at[b]], bufs[b], sems[b])

        def outer(o, carry):
            for b in range(NBUF):
                j = o * NBUF + b
                pltpu.make_async_copy(table_hbm.at[idx_v.at[b]], bufs[b],
                                      sems[b]).wait()
                pltpu.sync_copy(bufs[b],
                                out_hbm.at[pl.ds(base + j * CHUNK, CHUNK)])
                jn = j + NBUF

                @pl.when(jn < n_chunks)
                def _():
                    pltpu.async_copy(table_hbm.at[idx_v.at[jn]], bufs[b],
                                     sems[b])
            return carry

        lax.fori_loop(0, n_chunks // NBUF, outer, 0)

    run = pl.kernel(
        body,
        out_type=jax.ShapeDtypeStruct((B, D), table.dtype),
        mesh=mesh,
        scratch_types=dict(
            idx_v=pltpu.VMEM((n_chunks, CHUNK), jnp.int32),
            bufs=[pltpu.VMEM((CHUNK, D), table.dtype) for _ in range(NBUF)],
            sems=[pltpu.SemaphoreType.DMA for _ in range(NBUF)],
        ),
    )
    return run(table, idx3)


def _main_body(ea_ref, g_ref, eeW0, eeb0, eeW1, eeb1, eeW2, eeb2, eeg, eebe,
               epW0c, epb0, epW1, epb1, epW2, epb2, epg, epbe,
               npW0b, np_b0, np_W1, np_b1, np_W2, np_b2, np_g, np_be,
               decW0, decb0, decW1, decb1, decW2, decb2, out_ref):
    f32 = jnp.float32
    bl = ea_ref.shape[1]
    be = DEG * bl
    ea = ea_ref[...].reshape(be, 2)
    h = jnp.maximum(jnp.dot(ea, eeW0[...],
                            preferred_element_type=f32) + eeb0[...], 0.0)
    h = jnp.maximum(jnp.dot(h, eeW1[...],
                            preferred_element_type=f32) + eeb1[...], 0.0)
    e0 = jnp.dot(h, eeW2[...], preferred_element_type=f32) + eeb2[...]
    e0 = _ln(e0, eeg[...], eebe[...])
    ec = jnp.dot(e0, epW0c[...], preferred_element_type=f32) + epb0[...]
    g = g_ref[...]
    bs = g.shape[0] // DEG
    for b in range(bs):
        gb = g[b * DEG:(b + 1) * DEG].reshape(be, HID)
        hb = jnp.maximum(gb + ec, 0.0)
        hb = jnp.maximum(jnp.dot(hb, epW1[...],
                                 preferred_element_type=f32) + epb1[...], 0.0)
        eb = jnp.dot(hb, epW2[...], preferred_element_type=f32) + epb2[...]
        eb = _ln(eb, epg[...], epbe[...]) + e0
        agg = eb.reshape(DEG, bl, HID).sum(axis=0)
        nh = jnp.maximum(jnp.dot(agg, npW0b[...],
                                 preferred_element_type=f32) + np_b0[...], 0.0)
        nh = jnp.maximum(jnp.dot(nh, np_W1[...],
                                 preferred_element_type=f32) + np_b1[...], 0.0)
        xb = jnp.dot(nh, np_W2[...], preferred_element_type=f32) + np_b2[...]
        xb = _ln(xb, np_g[...], np_be[...])
        o = jnp.maximum(jnp.dot(xb, decW0[...],
                                preferred_element_type=f32) + decb0[...], 0.0)
        o = jnp.maximum(jnp.dot(o, decW1[...],
                                preferred_element_type=f32) + decb1[...], 0.0)
        o = jnp.dot(o, decW2[...], preferred_element_type=f32) + decb2[...]
        out_ref[b] = o


def kernel(processor_features, batch_size, edge_index, edge_attr,
           eeW0, eeb0, eeW1, eeb1, eeW2, eeb2, eeg, eebe,
           epW0, epb0, epW1, epb1, epW2, epb2, epg, epbe,
           np_W0, np_b0, np_W1, np_b1, np_W2, np_b2, np_g, np_be,
           decW0, decb0, decW1, decb1, decW2, decb2):
    del batch_size
    pf = processor_features
    bs = pf.shape[0] // NUM_H3
    E0 = edge_attr.shape[0]
    num_latlons = E0 // DEG

    epW0a = epW0[:HID]
    prep = pl.pallas_call(
        _prep_body,
        out_shape=jax.ShapeDtypeStruct((bs * NUM_H3, HID), jnp.float32),
    )
    table = prep(pf, epW0a)

    src = jnp.transpose(edge_index[0].reshape(num_latlons, DEG)).reshape(-1)
    ea = jnp.transpose(edge_attr.reshape(num_latlons, DEG, 2), (1, 0, 2))
    idx = jnp.concatenate([src + b * NUM_H3 for b in range(bs)])
    n_chunks = idx.shape[0] // (NW * CHUNK)
    idx3 = idx.reshape(NW, n_chunks, CHUNK)
    g = _sc_gather(table, idx3).reshape(bs * DEG, num_latlons, HID)

    BL = 1024
    grid = (num_latlons // BL,)
    full = lambda a: pl.BlockSpec(a.shape, lambda i: (0,) * a.ndim)
    weights = (eeW0, eeb0, eeW1, eeb1, eeW2, eeb2, eeg, eebe,
               epW0[2 * HID:], epb0, epW1, epb1, epW2, epb2, epg, epbe,
               np_W0[HID:], np_b0, np_W1, np_b1, np_W2, np_b2, np_g, np_be,
               decW0, decb0, decW1, decb1, decW2, decb2)
    out = pl.pallas_call(
        _main_body,
        grid=grid,
        in_specs=[
            pl.BlockSpec((DEG, BL, 2), lambda i: (0, i, 0)),
            pl.BlockSpec((bs * DEG, BL, HID), lambda i: (0, i, 0)),
        ] + [full(w) for w in weights],
        out_specs=pl.BlockSpec((bs, BL, OUT_DIM), lambda i: (0, i, 0)),
        out_shape=jax.ShapeDtypeStruct((bs, num_latlons, OUT_DIM),
                                       jnp.float32),
        compiler_params=pltpu.CompilerParams(
            dimension_semantics=("arbitrary",)),
    )(ea, g, *weights)
    return out


# speedup vs baseline: 8.7356x; 1.1027x over previous
import functools

import jax
import jax.numpy as jnp
from jax import lax
from jax.experimental import pallas as pl
from jax.experimental.pallas import tpu as pltpu
from jax.experimental.pallas import tpu_sc as plsc

NUM_H3 = 5882
HID = 128
OUT_DIM = 78
DEG = 7

NW = 32
CHUNK = 128
NBUF = 4


def _ln(x, g, b, eps=1e-5):
    m = jnp.mean(x, axis=-1, keepdims=True)
    xc = x - m
    v = jnp.mean(xc * xc, axis=-1, keepdims=True)
    return xc * lax.rsqrt(v + eps) * g + b


def _prep_body(pf_ref, w_ref, out_ref):
    out_ref[...] = jnp.dot(pf_ref[...], w_ref[...],
                           preferred_element_type=jnp.float32)


def _sc_gather(table, idx3):
    n_chunks = idx3.shape[1]
    b_per_w = n_chunks * CHUNK
    B = NW * b_per_w
    D = table.shape[1]
    mesh = plsc.VectorSubcoreMesh(core_axis_name="c", subcore_axis_name="s")

    def body(table_hbm, idx_hbm, out_hbm, idx_v, bufs, sems):
        wid = lax.axis_index("s") * 2 + lax.axis_index("c")
        base = wid * b_per_w
        pltpu.sync_copy(idx_hbm.at[wid], idx_v)
        for b in range(NBUF):
            pltpu.async_copy(table_hbm.at[idx_v.at[b]], bufs[b], sems[b])

        def outer(o, carry):
            for b in range(NBUF):
                j = o * NBUF + b
                pltpu.make_async_copy(table_hbm.at[idx_v.at[b]], bufs[b],
                                      sems[b]).wait()
                pltpu.sync_copy(bufs[b],
                                out_hbm.at[pl.ds(base + j * CHUNK, CHUNK)])
                jn = j + NBUF

                @pl.when(jn < n_chunks)
                def _():
                    pltpu.async_copy(table_hbm.at[idx_v.at[jn]], bufs[b],
                                     sems[b])
            return carry

        lax.fori_loop(0, n_chunks // NBUF, outer, 0)

    run = pl.kernel(
        body,
        out_type=jax.ShapeDtypeStruct((B, D), table.dtype),
        mesh=mesh,
        scratch_types=dict(
            idx_v=pltpu.VMEM((n_chunks, CHUNK), jnp.int32),
            bufs=[pltpu.VMEM((CHUNK, D), table.dtype) for _ in range(NBUF)],
            sems=[pltpu.SemaphoreType.DMA for _ in range(NBUF)],
        ),
        compiler_params=pltpu.CompilerParams(use_tc_tiling_on_sc=True),
    )
    return run(table, idx3)


def _main_body(ea_ref, g_ref, eeW0, eeb0, eeW1, eeb1, eeW2, eeb2, eeg, eebe,
               epW0c, epb0, epW1, epb1, epW2, epb2, epg, epbe,
               npW0b, np_b0, np_W1, np_b1, np_W2, np_b2, np_g, np_be,
               decW0, decb0, decW1, decb1, decW2, decb2, out_ref):
    f32 = jnp.float32
    bl = ea_ref.shape[0]
    be = DEG * bl
    ea = ea_ref[...]
    b0 = eeb0[...]
    h = jnp.concatenate(
        [jnp.maximum(jnp.dot(ea, eeW0[k], preferred_element_type=jnp.float32)
                     + b0, 0.0) for k in range(DEG)], axis=0)
    h = jnp.maximum(jnp.dot(h, eeW1[...],
                            preferred_element_type=f32) + eeb1[...], 0.0)
    e0 = jnp.dot(h, eeW2[...], preferred_element_type=f32) + eeb2[...]
    e0 = _ln(e0, eeg[...], eebe[...])
    ec = jnp.dot(e0, epW0c[...], preferred_element_type=f32) + epb0[...]
    g = g_ref[...]
    bs = g.shape[0] // DEG
    for b in range(bs):
        gb = g[b * DEG:(b + 1) * DEG].reshape(be, HID)
        hb = jnp.maximum(gb + ec, 0.0)
        hb = jnp.maximum(jnp.dot(hb, epW1[...],
                                 preferred_element_type=f32) + epb1[...], 0.0)
        eb = jnp.dot(hb, epW2[...], preferred_element_type=f32) + epb2[...]
        eb = _ln(eb, epg[...], epbe[...]) + e0
        agg = eb.reshape(DEG, bl, HID).sum(axis=0)
        nh = jnp.maximum(jnp.dot(agg, npW0b[...],
                                 preferred_element_type=f32) + np_b0[...], 0.0)
        nh = jnp.maximum(jnp.dot(nh, np_W1[...],
                                 preferred_element_type=f32) + np_b1[...], 0.0)
        xb = jnp.dot(nh, np_W2[...], preferred_element_type=f32) + np_b2[...]
        xb = _ln(xb, np_g[...], np_be[...])
        o = jnp.maximum(jnp.dot(xb, decW0[...],
                                preferred_element_type=f32) + decb0[...], 0.0)
        o = jnp.maximum(jnp.dot(o, decW1[...],
                                preferred_element_type=f32) + decb1[...], 0.0)
        o = jnp.dot(o, decW2[...], preferred_element_type=f32) + decb2[...]
        out_ref[b] = o


def kernel(processor_features, batch_size, edge_index, edge_attr,
           eeW0, eeb0, eeW1, eeb1, eeW2, eeb2, eeg, eebe,
           epW0, epb0, epW1, epb1, epW2, epb2, epg, epbe,
           np_W0, np_b0, np_W1, np_b1, np_W2, np_b2, np_g, np_be,
           decW0, decb0, decW1, decb1, decW2, decb2):
    del batch_size
    pf = processor_features
    bs = pf.shape[0] // NUM_H3
    E0 = edge_attr.shape[0]
    num_latlons = E0 // DEG

    epW0a = epW0[:HID]
    prep = pl.pallas_call(
        _prep_body,
        out_shape=jax.ShapeDtypeStruct((bs * NUM_H3, HID), jnp.float32),
    )
    table = prep(pf, epW0a)

    src = jnp.transpose(edge_index[0].reshape(num_latlons, DEG)).reshape(-1)
    ea = edge_attr.reshape(num_latlons, 2 * DEG)
    idx = jnp.concatenate([src + b * NUM_H3 for b in range(bs)])
    n_chunks = idx.shape[0] // (NW * CHUNK)
    idx3 = idx.reshape(NW, n_chunks, CHUNK)
    g = _sc_gather(table, idx3).reshape(bs * DEG, num_latlons, HID)

    BL = 1024
    grid = (num_latlons // BL,)
    full = lambda a: pl.BlockSpec(a.shape, lambda i: (0,) * a.ndim)
    ee_w0 = jnp.zeros((DEG, 2 * DEG, HID), jnp.float32)
    for k in range(DEG):
        ee_w0 = ee_w0.at[k, 2 * k].set(eeW0[0]).at[k, 2 * k + 1].set(eeW0[1])
    weights = (ee_w0, eeb0, eeW1, eeb1, eeW2, eeb2, eeg, eebe,
               epW0[2 * HID:], epb0, epW1, epb1, epW2, epb2, epg, epbe,
               np_W0[HID:], np_b0, np_W1, np_b1, np_W2, np_b2, np_g, np_be,
               decW0, decb0, decW1, decb1, decW2, decb2)
    out = pl.pallas_call(
        _main_body,
        grid=grid,
        in_specs=[
            pl.BlockSpec((BL, 2 * DEG), lambda i: (i, 0)),
            pl.BlockSpec((bs * DEG, BL, HID), lambda i: (0, i, 0)),
        ] + [full(w) for w in weights],
        out_specs=pl.BlockSpec((bs, BL, OUT_DIM), lambda i: (0, i, 0)),
        out_shape=jax.ShapeDtypeStruct((bs, num_latlons, OUT_DIM),
                                       jnp.float32),
        compiler_params=pltpu.CompilerParams(
            dimension_semantics=("arbitrary",)),
    )(ea, g, *weights)
    return out
